# baseline (speedup 1.0000x reference)
import functools

import jax
import jax.numpy as jnp
from jax import lax
from jax.experimental import pallas as pl
from jax.experimental.pallas import tpu as pltpu
from jax.experimental.pallas import tpu_sc as plsc

SC_ROWS = 2048


def kernel(x, table):
    S, D = table.shape
    info = plsc.get_sparse_core_info()
    NC, NS = info.num_cores, info.num_subcores
    NW = NC * NS
    rows_per_w = SC_ROWS // NW

    mesh = plsc.VectorSubcoreMesh(core_axis_name="c", subcore_axis_name="s")

    @functools.partial(
        pl.kernel,
        mesh=mesh,
        out_type=jax.ShapeDtypeStruct((SC_ROWS, D), jnp.float32),
        scratch_types=[
            pltpu.VMEM((rows_per_w, D), jnp.float32),
            pltpu.SemaphoreType.DMA,
        ],
    )
    def sc_copy(table_hbm, out_hbm, buf, sem):
        wid = lax.axis_index("s") * NC + lax.axis_index("c")
        base = wid * rows_per_w
        pltpu.async_copy(table_hbm.at[pl.ds(base, rows_per_w)], buf, sem).wait()
        pltpu.async_copy(buf, out_hbm.at[pl.ds(base, rows_per_w)], sem).wait()

    lo = sc_copy(table[:SC_ROWS])

    TC_ROWS = S - SC_ROWS

    def body(t_ref, o_ref):
        o_ref[...] = t_ref[...]

    hi = pl.pallas_call(
        body,
        out_shape=jax.ShapeDtypeStruct((TC_ROWS, D), jnp.float32),
        grid=(6,),
        in_specs=[pl.BlockSpec((TC_ROWS // 6, D), lambda i: (i, 0))],
        out_specs=pl.BlockSpec((TC_ROWS // 6, D), lambda i: (i, 0)),
    )(table[SC_ROWS:])

    return jnp.concatenate([lo, hi], axis=0)[None]


# speedup vs baseline: 1.4084x; 1.4084x over previous
import functools

import jax
import jax.numpy as jnp
from jax import lax
from jax.experimental import pallas as pl
from jax.experimental.pallas import tpu as pltpu
from jax.experimental.pallas import tpu_sc as plsc

NCHUNK = 4


def kernel(x, table):
    S, D = table.shape
    info = plsc.get_sparse_core_info()
    NC, NS = info.num_cores, info.num_subcores
    NW = NC * NS
    rows_per_w = S // NW
    chunk = rows_per_w // NCHUNK

    mesh = plsc.VectorSubcoreMesh(core_axis_name="c", subcore_axis_name="s")

    @functools.partial(
        pl.kernel,
        mesh=mesh,
        out_type=jax.ShapeDtypeStruct((S, D), jnp.float32),
        scratch_types=[
            pltpu.VMEM((NCHUNK, chunk, D), jnp.float32),
            pltpu.SemaphoreType.DMA,
            pltpu.SemaphoreType.DMA,
        ],
    )
    def copy_k(table_hbm, out_hbm, buf, sem_in, sem_out):
        wid = lax.axis_index("s") * NC + lax.axis_index("c")
        base = wid * rows_per_w
        gathers = [
            pltpu.async_copy(
                table_hbm.at[pl.ds(base + i * chunk, chunk)], buf.at[i], sem_in)
            for i in range(NCHUNK)
        ]
        scatters = []
        for i in range(NCHUNK):
            gathers[i].wait()
            scatters.append(pltpu.async_copy(
                buf.at[i], out_hbm.at[pl.ds(base + i * chunk, chunk)], sem_out))
        for s in scatters:
            s.wait()

    return copy_k(table)[None]


# speedup vs baseline: 1.4200x; 1.0082x over previous
import functools

import jax
import jax.numpy as jnp
from jax import lax
from jax.experimental import pallas as pl
from jax.experimental.pallas import tpu as pltpu
from jax.experimental.pallas import tpu_sc as plsc

NCHUNK = 2


def kernel(x, table):
    S, D = table.shape
    info = plsc.get_sparse_core_info()
    NC, NS = info.num_cores, info.num_subcores
    NW = NC * NS
    rows_per_w = S // NW
    chunk = rows_per_w // NCHUNK

    mesh = plsc.VectorSubcoreMesh(core_axis_name="c", subcore_axis_name="s")

    @functools.partial(
        pl.kernel,
        mesh=mesh,
        out_type=jax.ShapeDtypeStruct((S, D), jnp.float32),
        scratch_types=[
            pltpu.VMEM((NCHUNK, chunk, D), jnp.float32),
            pltpu.SemaphoreType.DMA,
            pltpu.SemaphoreType.DMA,
        ],
    )
    def copy_k(table_hbm, out_hbm, buf, sem_in, sem_out):
        wid = lax.axis_index("s") * NC + lax.axis_index("c")
        base = wid * rows_per_w
        gathers = [
            pltpu.async_copy(
                table_hbm.at[pl.ds(base + i * chunk, chunk)], buf.at[i], sem_in)
            for i in range(NCHUNK)
        ]
        scatters = []
        for i in range(NCHUNK):
            gathers[i].wait()
            scatters.append(pltpu.async_copy(
                buf.at[i], out_hbm.at[pl.ds(base + i * chunk, chunk)], sem_out))
        for s in scatters:
            s.wait()

    return copy_k(table)[None]
